# baseline (speedup 1.0000x reference)
import functools

import jax
import jax.numpy as jnp
from jax import lax
from jax.experimental import pallas as pl
from jax.experimental.pallas import tpu as pltpu
from jax.experimental.pallas import tpu_sc as plsc

VOCAB = 1000
VPAD = 1024
HEAD = 896
NC = 2
NS = 16
NW = NC * NS


def _sc_gather(idx4, table_p, b, t, tp):
    nb = b // NW
    mesh = plsc.VectorSubcoreMesh(core_axis_name="c", subcore_axis_name="s")

    @functools.partial(
        pl.kernel,
        mesh=mesh,
        out_type=jax.ShapeDtypeStruct((b, t, VOCAB), jnp.float32),
        scratch_types=[
            pltpu.VMEM((1, tp), jnp.int32),
            pltpu.VMEM((tp, VPAD), jnp.float32),
            pltpu.VMEM((t, VOCAB), jnp.float32),
            pltpu.SemaphoreType.DMA,
        ],
    )
    def k(idx_hbm, table_hbm, out_hbm, idx_v, gbuf, abuf, gsem):
        wid = lax.axis_index("s") * NC + lax.axis_index("c")

        def body(g, _):
            bb = wid * nb + g
            pltpu.sync_copy(idx_hbm.at[bb], idx_v)
            pltpu.async_copy(table_hbm.at[idx_v.at[0]], gbuf, gsem).wait()

            def stitch(r, _):
                for off in [16 * j for j in range(62)] + [VOCAB - 16]:
                    abuf[r, pl.ds(off, 16)] = gbuf[r, pl.ds(off, 16)]
                return 0

            lax.fori_loop(0, t, stitch, 0)
            pltpu.sync_copy(abuf, out_hbm.at[bb])
            return 0

        lax.fori_loop(0, nb, body, 0)

    return k(idx4, table_p)


def kernel(idx, table):
    b, t = idx.shape
    tp = (t + 7) // 8 * 8
    idx4 = jnp.pad(idx.reshape(b, 1, t), ((0, 0), (0, 0), (0, tp - t)))
    table_p = jnp.pad(table, ((0, 0), (0, VPAD - VOCAB)))
    return _sc_gather(idx4, table_p, b, t, tp)
